# Initial kernel scaffold; baseline (speedup 1.0000x reference)
import jax
import jax.numpy as jnp
from jax.experimental import pallas as pl


def kernel(x_s, x_t, edge_attr, edge_index, params):
    raise NotImplementedError("write your pallas kernel here")


# speedup vs baseline: 2.1554x; 2.1554x over previous
import functools

import numpy as np
import jax
import jax.numpy as jnp
from jax import lax
from jax.experimental import pallas as pl
from jax.experimental.pallas import tpu as pltpu
from jax.experimental.pallas import tpu_sc as plsc

F32 = jnp.float32
H = 128
N_NODES = 10000
N_EDGES = 320000
NUM_LAYERS = 2

NC = 2
NS = 16
NW = NC * NS
EW = N_EDGES // NW
CHUNK = 80
NCHUNK = EW // CHUNK
ROWS_PER_TILE = 624
TAIL_ROWS = N_NODES - NS * ROWS_PER_TILE


_sc_mesh = plsc.VectorSubcoreMesh(core_axis_name="c", subcore_axis_name="s")


@functools.partial(
    pl.kernel,
    out_type=jax.ShapeDtypeStruct((NC, N_NODES, H), F32),
    mesh=_sc_mesh,
    scratch_types=[
        pltpu.VMEM((CHUNK,), jnp.int32),
        pltpu.VMEM((CHUNK,), jnp.int32),
        pltpu.VMEM((CHUNK,), F32),
        pltpu.VMEM((CHUNK, H), F32),
        pltpu.VMEM((CHUNK, H), F32),
        pltpu.VMEM((CHUNK, H), F32),
        pltpu.VMEM((H,), F32),
        pltpu.VMEM_SHARED((N_NODES, H), F32),
        pltpu.SemaphoreType.DMA,
        pltpu.SemaphoreType.DMA,
    ],
)
def _sc_edge_kernel(a_hbm, b_hbm, src_hbm, dst_hbm, ea_hbm, w_hbm, zeros_hbm,
                    out_hbm, idx_s, idx_d, ea_v, a_v, b_v, msg, w_v, acc,
                    sem_a, sem_b):
    cid = lax.axis_index("c")
    sid = lax.axis_index("s")
    wid = sid * NC + cid

    r0 = sid * ROWS_PER_TILE
    pltpu.sync_copy(zeros_hbm.at[pl.ds(r0, ROWS_PER_TILE)],
                    acc.at[pl.ds(r0, ROWS_PER_TILE)])

    @pl.when(sid == 0)
    def _zero_tail():
        pltpu.sync_copy(zeros_hbm.at[pl.ds(NS * ROWS_PER_TILE, TAIL_ROWS)],
                        acc.at[pl.ds(NS * ROWS_PER_TILE, TAIL_ROWS)])
    pltpu.sync_copy(w_hbm, w_v)
    ws = [w_v[pl.ds(16 * k, 16)] for k in range(8)]
    plsc.subcore_barrier()

    base0 = wid * EW

    def chunk_body(c, carry):
        base = base0 + c * CHUNK
        pltpu.sync_copy(src_hbm.at[pl.ds(base, CHUNK)], idx_s)
        pltpu.sync_copy(dst_hbm.at[pl.ds(base, CHUNK)], idx_d)
        pltpu.sync_copy(ea_hbm.at[pl.ds(base, CHUNK)], ea_v)
        cp_a = pltpu.async_copy(a_hbm.at[idx_d], a_v, sem_a)
        cp_b = pltpu.async_copy(b_hbm.at[idx_s], b_v, sem_b)
        cp_a.wait()
        cp_b.wait()

        def group_body(g, carry2):
            ea16 = ea_v[pl.ds(16 * g, 16)]
            for jj in range(16):
                j = 16 * g + jj
                e = ea16[jj]
                for k in range(8):
                    sl = pl.ds(16 * k, 16)
                    m = a_v[j, sl] + b_v[j, sl] + e * ws[k]
                    msg[j, sl] = jnp.maximum(m, 0.0)
            return carry2

        lax.fori_loop(0, CHUNK // 16, group_body, 0)
        pltpu.sync_copy(msg, acc.at[idx_d], add=True)
        return carry

    lax.fori_loop(0, NCHUNK, chunk_body, 0)
    plsc.subcore_barrier()
    pltpu.sync_copy(acc.at[pl.ds(r0, ROWS_PER_TILE)],
                    out_hbm.at[cid, pl.ds(r0, ROWS_PER_TILE)])

    @pl.when(sid == 0)
    def _write_tail():
        pltpu.sync_copy(acc.at[pl.ds(NS * ROWS_PER_TILE, TAIL_ROWS)],
                        out_hbm.at[cid, pl.ds(NS * ROWS_PER_TILE, TAIL_ROWS)])


BLK = 2000
GRID = N_NODES // BLK

_row_spec = pl.BlockSpec((BLK, H), lambda i: (i, 0))
_w_spec = pl.BlockSpec((H, H), lambda i: (0, 0))
_b_spec = pl.BlockSpec((1, H), lambda i: (0, 0))


def _fourier_body(x_ref, sc_ref, ph_ref, o_ref):
    o_ref[...] = jnp.sin(x_ref[...] * sc_ref[...] + ph_ref[...])


_fourier_call = pl.pallas_call(
    _fourier_body,
    grid=(GRID,),
    in_specs=[_row_spec, _b_spec, _b_spec],
    out_specs=_row_spec,
    out_shape=jax.ShapeDtypeStruct((N_NODES, H), F32),
)


def _pre_body(l_ref, r_ref, wlt, bl, wrt, a_ref, b_ref):
    a_ref[...] = jnp.dot(r_ref[...], wlt[...], preferred_element_type=F32) + bl[...]
    b_ref[...] = jnp.dot(l_ref[...], wrt[...], preferred_element_type=F32)


_pre_call = pl.pallas_call(
    _pre_body,
    grid=(GRID,),
    in_specs=[_row_spec, _row_spec, _w_spec, _b_spec, _w_spec],
    out_specs=[_row_spec, _row_spec],
    out_shape=[jax.ShapeDtypeStruct((N_NODES, H), F32)] * 2,
)


def _post_body(s_ref, r_ref, wft, w1at, w1bt, b1, w2t, b2, g, b, o_ref):
    s = s_ref[0] + s_ref[1]
    agg = jnp.dot(s, wft[...], preferred_element_type=F32)
    u = (jnp.dot(agg, w1at[...], preferred_element_type=F32)
         + jnp.dot(r_ref[...], w1bt[...], preferred_element_type=F32) + b1[...])
    u = jnp.maximum(u, 0.0)
    h = jnp.dot(u, w2t[...], preferred_element_type=F32) + b2[...]
    mu = jnp.mean(h, axis=-1, keepdims=True)
    d = h - mu
    var = jnp.mean(d * d, axis=-1, keepdims=True)
    o_ref[...] = d * lax.rsqrt(var + 1e-5) * g[...] + b[...]


_post_call = pl.pallas_call(
    _post_body,
    grid=(GRID,),
    in_specs=[pl.BlockSpec((NC, BLK, H), lambda i: (0, i, 0)), _row_spec,
              _w_spec, _w_spec, _w_spec, _b_spec, _w_spec, _b_spec,
              _b_spec, _b_spec],
    out_specs=_row_spec,
    out_shape=jax.ShapeDtypeStruct((N_NODES, H), F32),
)

_cols = np.arange(H)
_F_SCALE = ((2.0 ** (_cols // 8)) * np.pi).reshape(1, H).astype(np.float32)
_F_PHASE = np.where((_cols % 8) // 4 == 0, 0.0,
                    np.pi / 2).reshape(1, H).astype(np.float32)


def _conv_apply(p, left, right, src, dst, ea, zeros):
    pre = p['pre_scale']
    wlt = p['W_left'].T * pre
    blv = (p['b_left'] * pre).reshape(1, H)
    wrt = p['W_right'].T * pre
    wvec = p['W_edge'][:, 0] * pre
    a_tab, b_tab = _pre_call(left, right, wlt, blv, wrt)
    s2 = _sc_edge_kernel(a_tab, b_tab, src, dst, ea, wvec, zeros)
    wft = p['W_final'].T
    w1at = p['W1'][:, :H].T * p['post_scale']
    w1bt = p['W1'][:, H:].T
    return _post_call(s2, right, wft, w1at, w1bt, p['b1'].reshape(1, H),
                      p['W2'].T, p['b2'].reshape(1, H),
                      p['ln_g'].reshape(1, H), p['ln_b'].reshape(1, H))


def kernel(x_s, x_t, edge_attr, edge_index, params):
    src = edge_index[0]
    dst = edge_index[1]
    ea = edge_attr[:, 0]
    zeros = jnp.zeros((N_NODES, H), F32)
    fsc = jnp.asarray(_F_SCALE)
    fph = jnp.asarray(_F_PHASE)
    xs = _fourier_call(jnp.tile(x_s, (1, H // x_s.shape[1])), fsc, fph)
    xt = _fourier_call(jnp.tile(x_t, (1, H // x_t.shape[1])), fsc, fph)
    xs_outs, xt_outs = [], []
    for l in range(NUM_LAYERS):
        xt = _conv_apply(params['s_t'][l], xs, xt, src, dst, ea, zeros)
        xs = _conv_apply(params['t_s'][l], xt, xs, dst, src, ea, zeros)
        xs_outs.append(xs)
        xt_outs.append(xt)
    return (jnp.concatenate(xs_outs, axis=-1), jnp.concatenate(xt_outs, axis=-1))
